# Initial kernel scaffold; baseline (speedup 1.0000x reference)
import jax
import jax.numpy as jnp
from jax.experimental import pallas as pl


def kernel(input, weight):
    raise NotImplementedError("write your pallas kernel here")


# speedup vs baseline: 5.0500x; 5.0500x over previous
import functools

import jax
import jax.numpy as jnp
from jax import lax
from jax.experimental import pallas as pl
from jax.experimental.pallas import tpu as pltpu
from jax.experimental.pallas import tpu_sc as plsc

_EMB = 32
_BATCH = 16384
_HIST = 200
_B = _BATCH * _HIST
_NW = 32
_BPW = _B // _NW
_C = 800
_G = _BPW // _C
_NBUF = 4
_K = 2

_mesh = plsc.VectorSubcoreMesh(core_axis_name="c", subcore_axis_name="s")


@functools.partial(
    pl.kernel,
    mesh=_mesh,
    out_type=jax.ShapeDtypeStruct((_B, _EMB), jnp.float32),
    scratch_types=[
        pltpu.VMEM((_NBUF * _C,), jnp.int32),
        pltpu.VMEM((_NBUF, _C, _EMB), jnp.float32),
    ] + [pltpu.SemaphoreType.DMA] * (2 * _NBUF),
    compiler_params=pltpu.CompilerParams(use_tc_tiling_on_sc=False),
)
def _emb_lookup(idx_hbm, table_hbm, out_hbm, idx_v, rows_v, *sems):
    sem_g = sems[:_NBUF]
    sem_w = sems[_NBUF:]
    wid = lax.axis_index("s") * 2 + lax.axis_index("c")
    base = wid * _BPW

    def fire(j, bj):
        pltpu.sync_copy(idx_hbm.at[pl.ds(base + j * _C, _C)], idx_v.at[pl.ds(bj * _C, _C)])
        pltpu.async_copy(table_hbm.at[idx_v.at[pl.ds(bj * _C, _C)]], rows_v.at[bj], sem_g[bj])

    for j in range(_K):
        fire(j, j)

    def group(gg, carry):
        for phase in range(_NBUF):
            i = gg * _NBUF + phase
            bi = phase
            bj = (phase + _K) % _NBUF
            j = i + _K

            @pl.when(j < _G)
            def _():
                @pl.when(j >= _NBUF)
                def _():
                    pltpu.make_async_copy(
                        rows_v.at[bj], out_hbm.at[pl.ds(base, _C)], sem_w[bj]
                    ).wait()

                fire(j, bj)

            pltpu.make_async_copy(
                table_hbm.at[idx_v.at[pl.ds(bi * _C, _C)]], rows_v.at[bi], sem_g[bi]
            ).wait()
            pltpu.async_copy(
                rows_v.at[bi], out_hbm.at[pl.ds(base + i * _C, _C)], sem_w[bi]
            )
        return carry

    lax.fori_loop(0, _G // _NBUF, group, 0)

    for b in range(_NBUF):
        pltpu.make_async_copy(
            rows_v.at[b], out_hbm.at[pl.ds(base, _C)], sem_w[b]
        ).wait()


def kernel(input, weight):
    idx = input.reshape(_B).astype(jnp.int32)
    out = _emb_lookup(idx, weight)
    return out.reshape(_BATCH, _HIST, _EMB)
